# baseline (speedup 1.0000x reference)
import jax
import jax.numpy as jnp
from jax.experimental import pallas as pl
from jax.experimental.pallas import tpu as pltpu

_ROWS = 1024


def _lane_pair_max(h):
    r, width = h.shape
    idx = jax.lax.broadcasted_iota(jnp.int32, (r, width // 2), 1) * 2
    e = jnp.take_along_axis(h, idx, axis=1)
    o = jnp.take_along_axis(h, idx + 1, axis=1)
    return jnp.maximum(e, o)


def _pool_body(x_ref, o_ref, s_ref):
    x = x_ref[...]
    s_ref[:, 0:64] = _lane_pair_max(x[:, :128])
    s_ref[:, 64:112] = _lane_pair_max(x[:, 128:])
    half = _ROWS // 2
    a = s_ref[pl.ds(0, half, 2), :]
    b = s_ref[pl.ds(1, half, 2), :]
    o_ref[...] = jnp.maximum(a, b)[:, :112]


def kernel(x):
    n, c, hh, ww = x.shape
    rows = n * c * hh
    x2 = x.reshape(rows, ww)
    grid = rows // _ROWS
    out = pl.pallas_call(
        _pool_body,
        grid=(grid,),
        in_specs=[pl.BlockSpec((_ROWS, ww), lambda i: (i, 0))],
        out_specs=pl.BlockSpec((_ROWS // 2, ww // 2), lambda i: (i, 0)),
        out_shape=jax.ShapeDtypeStruct((rows // 2, ww // 2), x.dtype),
        scratch_shapes=[pltpu.VMEM((_ROWS, 128), jnp.float32)],
        compiler_params=pltpu.CompilerParams(
            dimension_semantics=("parallel",),
        ),
    )(x2)
    return out.reshape(n, c, hh // 2, ww // 2)


# speedup vs baseline: 1.1256x; 1.1256x over previous
import jax
import jax.numpy as jnp
from jax.experimental import pallas as pl
from jax.experimental.pallas import tpu as pltpu

_B = 8


def _lane_pair_max(h):
    shape = h.shape[:-1] + (h.shape[-1] // 2,)
    idx = jax.lax.broadcasted_iota(jnp.int32, shape, len(shape) - 1) * 2
    e = jnp.take_along_axis(h, idx, axis=-1)
    o = jnp.take_along_axis(h, idx + 1, axis=-1)
    return jnp.maximum(e, o)


def _pool_body(x_ref, o_ref, s_ref):
    x = x_ref[...]
    s_ref[:, :, 0:64] = _lane_pair_max(x[:, :, :128])
    s_ref[:, :, 64:112] = _lane_pair_max(x[:, :, 128:])
    a = s_ref[:, pl.ds(0, 112, 2), :]
    b = s_ref[:, pl.ds(1, 112, 2), :]
    o_ref[...] = jnp.maximum(a, b)[:, :, :112]


def kernel(x):
    n, c, hh, ww = x.shape
    nc = n * c
    x3 = x.reshape(nc, hh, ww)
    grid = nc // _B
    out = pl.pallas_call(
        _pool_body,
        grid=(grid,),
        in_specs=[pl.BlockSpec((_B, hh, ww), lambda i: (i, 0, 0))],
        out_specs=pl.BlockSpec((_B, hh // 2, ww // 2), lambda i: (i, 0, 0)),
        out_shape=jax.ShapeDtypeStruct((nc, hh // 2, ww // 2), x.dtype),
        scratch_shapes=[pltpu.VMEM((_B, hh, 128), jnp.float32)],
        compiler_params=pltpu.CompilerParams(
            dimension_semantics=("parallel",),
        ),
    )(x3)
    return out.reshape(n, c, hh // 2, ww // 2)
